# Initial kernel scaffold; baseline (speedup 1.0000x reference)
import functools

import jax
import jax.numpy as jnp
from jax import lax
from jax.experimental import pallas as pl
from jax.experimental.pallas import tpu as pltpu
from jax.experimental.pallas import tpu_sc as plsc

NC = 2
NS = 16
NW = NC * NS

SUB = 128
SUBS_PER_CHUNK = 8
CHUNK = SUB * SUBS_PER_CHUNK


def _make_lookup(n, d):
    assert n % (NW * CHUNK) == 0, (n, NW * CHUNK)
    per_w = n // NW
    n_chunk = per_w // CHUNK
    mesh = plsc.VectorSubcoreMesh(core_axis_name="c", subcore_axis_name="s")

    @functools.partial(
        pl.kernel,
        mesh=mesh,
        out_type=jax.ShapeDtypeStruct((n, d), jnp.float32),
        scratch_types=[
            pltpu.VMEM((2, CHUNK), jnp.int32),
            pltpu.VMEM((2, CHUNK, d), jnp.float32),
            pltpu.SemaphoreType.DMA,
        ],
    )
    def lookup(idx_hbm, table_hbm, out_hbm, idx_v, rows_v, gsem):
        wid = lax.axis_index("s") * NC + lax.axis_index("c")
        base_w = wid * per_w

        def chunk_body(g, carry):
            base = base_w + g * CHUNK
            buf = g % 2
            pltpu.sync_copy(idx_hbm.at[pl.ds(base, CHUNK)], idx_v.at[buf])
            copies = []
            for j in range(SUBS_PER_CHUNK):
                copies.append(
                    pltpu.async_copy(
                        table_hbm.at[idx_v.at[buf].at[pl.ds(j * SUB, SUB)]],
                        rows_v.at[buf].at[pl.ds(j * SUB, SUB)],
                        gsem,
                    )
                )
            for c in copies:
                c.wait()
            pltpu.sync_copy(rows_v.at[buf], out_hbm.at[pl.ds(base, CHUNK)])
            return carry

        lax.fori_loop(0, n_chunk, chunk_body, 0)

    return lookup


def kernel(input, weight):
    b, h = input.shape
    v, d = weight.shape
    n = b * h
    idx = input.reshape(n).astype(jnp.int32)
    out = _make_lookup(n, d)(idx, weight)
    return out.reshape(b, h, d)


# speedup vs baseline: 4.8055x; 4.8055x over previous
import functools

import jax
import jax.numpy as jnp
from jax import lax
from jax.experimental import pallas as pl
from jax.experimental.pallas import tpu as pltpu
from jax.experimental.pallas import tpu_sc as plsc

NC = 2
NS = 16
NW = NC * NS

SUB = 128
SUBS_PER_CHUNK = 8
CHUNK = SUB * SUBS_PER_CHUNK


def _make_lookup(n, d):
    assert n % (NW * CHUNK) == 0, (n, NW * CHUNK)
    per_w = n // NW
    n_chunk = per_w // CHUNK
    mesh = plsc.VectorSubcoreMesh(core_axis_name="c", subcore_axis_name="s")

    @functools.partial(
        pl.kernel,
        mesh=mesh,
        out_type=jax.ShapeDtypeStruct((n, d), jnp.float32),
        compiler_params=pltpu.CompilerParams(use_tc_tiling_on_sc=False),
        scratch_types=[
            pltpu.VMEM((2, CHUNK), jnp.int32),
            pltpu.VMEM((2, CHUNK, d), jnp.float32),
            pltpu.SemaphoreType.DMA,
        ],
    )
    def lookup(idx_hbm, table_hbm, out_hbm, idx_v, rows_v, gsem):
        wid = lax.axis_index("s") * NC + lax.axis_index("c")
        base_w = wid * per_w

        def chunk_body(g, carry):
            base = base_w + g * CHUNK
            buf = g % 2
            pltpu.sync_copy(idx_hbm.at[pl.ds(base, CHUNK)], idx_v.at[buf])
            copies = []
            for j in range(SUBS_PER_CHUNK):
                copies.append(
                    pltpu.async_copy(
                        table_hbm.at[idx_v.at[buf].at[pl.ds(j * SUB, SUB)]],
                        rows_v.at[buf].at[pl.ds(j * SUB, SUB)],
                        gsem,
                    )
                )
            for c in copies:
                c.wait()
            pltpu.sync_copy(rows_v.at[buf], out_hbm.at[pl.ds(base, CHUNK)])
            return carry

        lax.fori_loop(0, n_chunk, chunk_body, 0)

    return lookup


def kernel(input, weight):
    b, h = input.shape
    v, d = weight.shape
    n = b * h
    idx = input.reshape(n).astype(jnp.int32)
    out = _make_lookup(n, d)(idx, weight)
    return out.reshape(b, h, d)


# speedup vs baseline: 5.0280x; 1.0463x over previous
import functools

import jax
import jax.numpy as jnp
from jax import lax
from jax.experimental import pallas as pl
from jax.experimental.pallas import tpu as pltpu
from jax.experimental.pallas import tpu_sc as plsc

NC = 2
NS = 16
NW = NC * NS

SUB = 128
SUBS_PER_CHUNK = 8
CHUNK = SUB * SUBS_PER_CHUNK


def _make_lookup(n, d):
    assert n % (NW * CHUNK) == 0, (n, NW * CHUNK)
    per_w = n // NW
    n_chunk = per_w // CHUNK
    mesh = plsc.VectorSubcoreMesh(core_axis_name="c", subcore_axis_name="s")

    @functools.partial(
        pl.kernel,
        mesh=mesh,
        out_type=jax.ShapeDtypeStruct((n, d), jnp.float32),
        compiler_params=pltpu.CompilerParams(use_tc_tiling_on_sc=False),
        scratch_types=[
            pltpu.VMEM((2, CHUNK), jnp.int32),
            pltpu.VMEM((2, CHUNK, d), jnp.float32),
            pltpu.SemaphoreType.DMA,
            pltpu.SemaphoreType.DMA,
            pltpu.SemaphoreType.DMA,
        ],
    )
    def lookup(idx_hbm, table_hbm, out_hbm, idx_v, rows_v, isem, gsem, osem):
        wid = lax.axis_index("s") * NC + lax.axis_index("c")
        base_w = wid * per_w

        def idx_copy(g, buf):
            return pltpu.make_async_copy(
                idx_hbm.at[pl.ds(base_w + g * CHUNK, CHUNK)], idx_v.at[buf],
                isem)

        def gather_copies(buf):
            return [
                pltpu.make_async_copy(
                    table_hbm.at[idx_v.at[buf].at[pl.ds(j * SUB, SUB)]],
                    rows_v.at[buf].at[pl.ds(j * SUB, SUB)],
                    gsem,
                )
                for j in range(SUBS_PER_CHUNK)
            ]

        def out_copy(g, buf):
            return pltpu.make_async_copy(
                rows_v.at[buf], out_hbm.at[pl.ds(base_w + g * CHUNK, CHUNK)],
                osem)

        first_idx = idx_copy(0, 0)
        first_idx.start()
        first_idx.wait()
        for c in gather_copies(0):
            c.start()
        idx_copy(1, 1).start()

        def body(g, carry):
            buf = g % 2
            nbuf = 1 - buf

            @pl.when(g < n_chunk - 1)
            def _():
                idx_copy(g + 1, nbuf).wait()

            for c in gather_copies(buf):
                c.wait()

            @pl.when(g >= 1)
            def _():
                out_copy(g - 1, nbuf).wait()

            @pl.when(g < n_chunk - 1)
            def _():
                for c in gather_copies(nbuf):
                    c.start()

            out_copy(g, buf).start()

            @pl.when(g < n_chunk - 2)
            def _():
                idx_copy(g + 2, buf).start()

            return carry

        lax.fori_loop(0, n_chunk, body, 0)
        out_copy(n_chunk - 1, (n_chunk - 1) % 2).wait()

    return lookup


def kernel(input, weight):
    b, h = input.shape
    v, d = weight.shape
    n = b * h
    idx = input.reshape(n).astype(jnp.int32)
    out = _make_lookup(n, d)(idx, weight)
    return out.reshape(b, h, d)


# speedup vs baseline: 5.0396x; 1.0023x over previous
import functools

import jax
import jax.numpy as jnp
from jax import lax
from jax.experimental import pallas as pl
from jax.experimental.pallas import tpu as pltpu
from jax.experimental.pallas import tpu_sc as plsc

NC = 2
NS = 16
NW = NC * NS

ROWS_PER_CHUNK = 8
SUB = 80


def _make_lookup(b, h, d):
    assert b % NW == 0, (b, NW)
    rows_w = b // NW
    assert rows_w % ROWS_PER_CHUNK == 0
    n_chunk = rows_w // ROWS_PER_CHUNK
    chunk = ROWS_PER_CHUNK * h
    assert chunk % SUB == 0 and SUB % 8 == 0
    subs = chunk // SUB
    mesh = plsc.VectorSubcoreMesh(core_axis_name="c", subcore_axis_name="s")

    @functools.partial(
        pl.kernel,
        mesh=mesh,
        out_type=jax.ShapeDtypeStruct((b, h, d), jnp.float32),
        compiler_params=pltpu.CompilerParams(use_tc_tiling_on_sc=False),
        scratch_types=[
            pltpu.VMEM((2, chunk), jnp.int32),
            pltpu.VMEM((2, chunk, d), jnp.float32),
            pltpu.SemaphoreType.DMA,
            pltpu.SemaphoreType.DMA,
            pltpu.SemaphoreType.DMA,
        ],
    )
    def lookup(idx_hbm, table_hbm, out_hbm, idx_v, rows_v, isem, gsem, osem):
        wid = lax.axis_index("s") * NC + lax.axis_index("c")
        row_w = wid * rows_w

        def idx_copy(g, buf):
            base = (row_w + g * ROWS_PER_CHUNK) * h
            return pltpu.make_async_copy(
                idx_hbm.at[pl.ds(base, chunk)], idx_v.at[buf], isem)

        def gather_copies(buf):
            return [
                pltpu.make_async_copy(
                    table_hbm.at[idx_v.at[buf].at[pl.ds(j * SUB, SUB)]],
                    rows_v.at[buf].at[pl.ds(j * SUB, SUB)],
                    gsem,
                )
                for j in range(subs)
            ]

        def out_copies(g, buf):
            row = row_w + g * ROWS_PER_CHUNK
            return [
                pltpu.make_async_copy(
                    rows_v.at[buf].at[pl.ds(r * h, h)],
                    out_hbm.at[row + r],
                    osem,
                )
                for r in range(ROWS_PER_CHUNK)
            ]

        first_idx = idx_copy(0, 0)
        first_idx.start()
        first_idx.wait()
        for c in gather_copies(0):
            c.start()
        idx_copy(1, 1).start()

        def body(g, carry):
            buf = g % 2
            nbuf = 1 - buf

            @pl.when(g < n_chunk - 1)
            def _():
                idx_copy(g + 1, nbuf).wait()

            for c in gather_copies(buf):
                c.wait()

            @pl.when(g >= 1)
            def _():
                for c in out_copies(g - 1, nbuf):
                    c.wait()

            @pl.when(g < n_chunk - 1)
            def _():
                for c in gather_copies(nbuf):
                    c.start()

            for c in out_copies(g, buf):
                c.start()

            @pl.when(g < n_chunk - 2)
            def _():
                idx_copy(g + 2, buf).start()

            return carry

        lax.fori_loop(0, n_chunk, body, 0)
        for c in out_copies(n_chunk - 1, (n_chunk - 1) % 2):
            c.wait()

    return lookup


def kernel(input, weight):
    b, h = input.shape
    v, d = weight.shape
    idx = input.reshape(b * h).astype(jnp.int32)
    return _make_lookup(b, h, d)(idx, weight)


# speedup vs baseline: 5.0540x; 1.0029x over previous
import functools

import jax
import jax.numpy as jnp
from jax import lax
from jax.experimental import pallas as pl
from jax.experimental.pallas import tpu as pltpu
from jax.experimental.pallas import tpu_sc as plsc

NC = 2
NS = 16
NW = NC * NS

ROWS_PER_CHUNK = 8
SUB = 80


def _make_lookup(b, h, d):
    assert b % NW == 0, (b, NW)
    rows_w = b // NW
    assert rows_w % ROWS_PER_CHUNK == 0
    n_chunk = rows_w // ROWS_PER_CHUNK
    chunk = ROWS_PER_CHUNK * h
    assert chunk % SUB == 0 and SUB % 8 == 0
    subs = chunk // SUB
    mesh = plsc.VectorSubcoreMesh(core_axis_name="c", subcore_axis_name="s")

    @functools.partial(
        pl.kernel,
        mesh=mesh,
        out_type=jax.ShapeDtypeStruct((b, h, d), jnp.float32),
        compiler_params=pltpu.CompilerParams(use_tc_tiling_on_sc=False),
        scratch_types=[
            pltpu.VMEM((2, chunk), jnp.int32),
            pltpu.VMEM((2, chunk, d), jnp.float32),
            pltpu.SemaphoreType.DMA,
            pltpu.SemaphoreType.DMA,
            pltpu.SemaphoreType.DMA,
        ],
    )
    def lookup(idx_hbm, table_hbm, out_hbm, idx_v, rows_v, isem, gsem, osem):
        wid = lax.axis_index("s") * NC + lax.axis_index("c")
        row_w = wid * rows_w

        def idx_copy(g, buf):
            base = (row_w + g * ROWS_PER_CHUNK) * h
            return pltpu.make_async_copy(
                idx_hbm.at[pl.ds(base, chunk)], idx_v.at[buf], isem)

        def gather_copies(buf):
            return [
                pltpu.make_async_copy(
                    table_hbm.at[idx_v.at[buf].at[pl.ds(j * SUB, SUB)]],
                    rows_v.at[buf].at[pl.ds(j * SUB, SUB)],
                    gsem,
                )
                for j in range(subs)
            ]

        def out_copies(g, buf):
            row = row_w + g * ROWS_PER_CHUNK
            return [
                pltpu.make_async_copy(
                    rows_v.at[buf].at[pl.ds(r * h, h)],
                    out_hbm.at[row + r],
                    osem,
                )
                for r in range(ROWS_PER_CHUNK)
            ]

        first_idx = idx_copy(0, 0)
        first_idx.start()
        first_idx.wait()
        for c in gather_copies(0):
            c.start()
        idx_copy(1, 1).start()

        def body(g, carry):
            buf = g % 2
            nbuf = 1 - buf

            @pl.when(g < n_chunk - 1)
            def _():
                idx_copy(g + 1, nbuf).wait()

            for c in gather_copies(buf):
                c.wait()

            @pl.when(g >= 1)
            def _():
                for c in out_copies(g - 1, nbuf):
                    c.wait()

            @pl.when(g < n_chunk - 1)
            def _():
                for c in gather_copies(nbuf):
                    c.start()

            for c in out_copies(g, buf):
                c.start()

            @pl.when(g < n_chunk - 2)
            def _():
                idx_copy(g + 2, buf).start()

            return carry

        lax.fori_loop(0, n_chunk, body, 0)
        for c in out_copies(n_chunk - 1, (n_chunk - 1) % 2):
            c.wait()

    return lookup


def kernel(input, weight):
    b, h = input.shape
    v, d = weight.shape
    half = b // 2
    lookup = _make_lookup(half, h, d)
    idx0 = input[:half].reshape(half * h).astype(jnp.int32)
    idx1 = input[half:].reshape(half * h).astype(jnp.int32)
    out0 = lookup(idx0, weight)
    out1 = lookup(idx1, weight)
    return jnp.concatenate([out0, out1], axis=0)
